# baseline (speedup 1.0000x reference)
import functools

import jax
import jax.numpy as jnp
from jax import lax
from jax.experimental import pallas as pl
from jax.experimental.pallas import tpu as pltpu
from jax.experimental.pallas import tpu_sc as plsc

B = 128
D = 2048
NS = 64
K = 8
H = D // 2

NW = 32
RPW = (B * K) // NW
BJ = 256
NJ = D // BJ
BB = 8
NB = B // BB

_NT = (((1,), (1,)), ((), ()))


def _dot_nt(a, b):
    return lax.dot_general(a, b, dimension_numbers=_NT,
                           preferred_element_type=jnp.float32)


def _dot_nt_bf16(a, b):
    return lax.dot_general(a.astype(jnp.bfloat16), b.astype(jnp.bfloat16),
                           dimension_numbers=_NT,
                           preferred_element_type=jnp.float32)


def _mr_body(x_ref, s_ref, w1_ref, b1_ref, w2_ref, b2_ref, tau_ref,
             copy_ref, idx_ref, w_ref, acc_ref):
    i = pl.program_id(0)
    s = s_ref[...]
    copy_ref[...] = s
    acc_ref[pl.ds(i * BB, BB), :] = jnp.mean(s, axis=1)

    @pl.when(i == NB - 1)
    def _():
        x = x_ref[...]
        m = acc_ref[...]
        w1 = w1_ref[...]
        h1 = _dot_nt(x, w1[:, :D]) + _dot_nt(m, w1[:, D:])
        h1 = h1 + b1_ref[...][None, :]
        h1 = 0.5 * h1 * (1.0 + lax.erf(h1 * (2.0 ** -0.5)))
        logits = _dot_nt(h1, w2_ref[...]) + b2_ref[...][None, :]
        logits = logits / (jnp.abs(tau_ref[0, 0]) + 0.1)

        iota = lax.broadcasted_iota(jnp.int32, (B, NS), 1)
        vals = logits
        top_v, top_i = [], []
        for _ in range(K):
            mx = jnp.max(vals, axis=1, keepdims=True)
            cand = jnp.where(vals == mx, iota, NS)
            sel = jnp.min(cand, axis=1, keepdims=True)
            top_v.append(mx)
            top_i.append(sel)
            vals = jnp.where(iota == sel, -jnp.inf, vals)
        vtop = jnp.concatenate(top_v, axis=1)
        e = jnp.exp(vtop - vtop[:, 0:1])
        w_ref[...] = e / jnp.sum(e, axis=1, keepdims=True)
        idx_ref[...] = jnp.concatenate(top_i, axis=1)


def _mean_route(x, S, W1, b1, W2, b2, tau):
    full = functools.partial(lambda n: (0,) * n)
    return pl.pallas_call(
        _mr_body,
        grid=(NB,),
        in_specs=[
            pl.BlockSpec((B, D), lambda i: (0, 0)),
            pl.BlockSpec((BB, NS, D), lambda i: (i, 0, 0)),
            pl.BlockSpec(W1.shape, lambda i: (0, 0)),
            pl.BlockSpec(b1.shape, lambda i: (0,)),
            pl.BlockSpec(W2.shape, lambda i: (0, 0)),
            pl.BlockSpec(b2.shape, lambda i: (0,)),
            pl.BlockSpec(memory_space=pltpu.SMEM),
        ],
        out_specs=(pl.BlockSpec((BB, NS, D), lambda i: (i, 0, 0)),
                   pl.BlockSpec((B, K), lambda i: (0, 0)),
                   pl.BlockSpec((B, K), lambda i: (0, 0))),
        out_shape=(jax.ShapeDtypeStruct((B, NS, D), jnp.float32),
                   jax.ShapeDtypeStruct((B, K), jnp.int32),
                   jax.ShapeDtypeStruct((B, K), jnp.float32)),
        scratch_shapes=[pltpu.VMEM((B, D), jnp.float32)],
    )(x, S, W1, b1, W2, b2, tau.reshape(1, 1))


def _sc_gather(table, gid):
    mesh = plsc.VectorSubcoreMesh(core_axis_name="c", subcore_axis_name="s")

    @functools.partial(
        pl.kernel,
        out_type=jax.ShapeDtypeStruct((B * K, D), jnp.float32),
        mesh=mesh,
        scratch_types=[
            pltpu.VMEM((RPW,), jnp.int32),
            pltpu.VMEM((RPW, D), jnp.float32),
            pltpu.SemaphoreType.DMA,
        ],
    )
    def gather_kernel(table_hbm, idx_hbm, out_hbm, idx_v, rows_v, sem):
        wid = lax.axis_index("s") * 2 + lax.axis_index("c")
        base = wid * RPW
        pltpu.sync_copy(idx_hbm.at[pl.ds(base, RPW)], idx_v)
        pltpu.async_copy(table_hbm.at[idx_v], rows_v, sem).wait()
        pltpu.sync_copy(rows_v, out_hbm.at[pl.ds(base, RPW)])

    return gather_kernel(table, gid)


def _gru_body(x_ref, h_ref, hb_ref, wih_ref, whh_ref, bih_ref, bhh_ref,
              wt_ref, upd_ref, mix_ref):
    x = x_ref[...]
    h = h_ref[...]
    wih = wih_ref[...]
    whh = whh_ref[...]

    def gi(g):
        r = _dot_nt_bf16(x, wih[g]) + bih_ref[...][g, 0, 0][None, :]
        return jnp.broadcast_to(r[None], (K, B, BJ)).reshape(K * B, BJ)

    def gh(g):
        return _dot_nt_bf16(h, whh[g]) + bhh_ref[...][g, 0, 0][None, :]

    r = jax.nn.sigmoid(gi(0) + gh(0))
    z = jax.nn.sigmoid(gi(1) + gh(1))
    n = jnp.tanh(gi(2) + r * gh(2))
    upd = (1.0 - z) * n + z * hb_ref[...]
    upd_ref[...] = upd.reshape(K, B, BJ)
    wt = wt_ref[...]
    mix_ref[...] = jnp.sum(upd.reshape(K, B, BJ) * wt[:, :, None], axis=0)


def _gru_mix(x, hrows, Wih3, Whh3, bih4, bhh4, wT):
    return pl.pallas_call(
        _gru_body,
        grid=(NJ,),
        in_specs=[
            pl.BlockSpec((B, D), lambda j: (0, 0)),
            pl.BlockSpec((K * B, D), lambda j: (0, 0)),
            pl.BlockSpec((K * B, BJ), lambda j: (0, j)),
            pl.BlockSpec((3, BJ, D), lambda j: (0, j, 0)),
            pl.BlockSpec((3, BJ, D), lambda j: (0, j, 0)),
            pl.BlockSpec((3, 1, 1, BJ), lambda j: (0, j, 0, 0)),
            pl.BlockSpec((3, 1, 1, BJ), lambda j: (0, j, 0, 0)),
            pl.BlockSpec((K, B), lambda j: (0, 0)),
        ],
        out_specs=(pl.BlockSpec((K, B, BJ), lambda j: (0, 0, j)),
                   pl.BlockSpec((B, BJ), lambda j: (0, j))),
        out_shape=(jax.ShapeDtypeStruct((K, B, D), jnp.float32),
                   jax.ShapeDtypeStruct((B, D), jnp.float32)),
    )(x, hrows, hrows, Wih3, Whh3, bih4, bhh4, wT)


def _proj_body(mix_ref, wv_ref, bv_ref, wo_ref, bo_ref, out_ref):
    v = _dot_nt_bf16(mix_ref[...], wv_ref[...]) + bv_ref[...][None, :]
    out_ref[...] = _dot_nt_bf16(v, wo_ref[...]) + bo_ref[...][None, :]


def _proj(mix, Wv, bv, Wo, bo):
    return pl.pallas_call(
        _proj_body,
        in_specs=[pl.BlockSpec(a.shape, functools.partial(lambda n: (0,) * n, a.ndim))
                  for a in (mix, Wv, bv, Wo, bo)],
        out_specs=pl.BlockSpec((B, D), lambda: (0, 0)),
        out_shape=jax.ShapeDtypeStruct((B, D), jnp.float32),
    )(mix, Wv, bv, Wo, bo)


def _scatter_body(gid_ref, upd_ref, base_ref, out_ref, sem):
    def issue(r, c):
        g = gid_ref[r]
        pltpu.make_async_copy(upd_ref.at[pl.ds(r, 1), :],
                              out_ref.at[pl.ds(g, 1), :], sem).start()
        return c

    lax.fori_loop(0, B * K, issue, 0)

    def drain(r, c):
        pltpu.make_async_copy(upd_ref.at[pl.ds(0, 1), :],
                              out_ref.at[pl.ds(0, 1), :], sem).wait()
        return c

    lax.fori_loop(0, B * K, drain, 0)


def _scatter(gid, upd_rows, s_copy_table):
    return pl.pallas_call(
        _scatter_body,
        in_specs=[
            pl.BlockSpec(memory_space=pltpu.MemorySpace.SMEM),
            pl.BlockSpec(memory_space=pl.ANY),
            pl.BlockSpec(memory_space=pl.ANY),
        ],
        out_specs=pl.BlockSpec(memory_space=pl.ANY),
        out_shape=jax.ShapeDtypeStruct((B * NS, D), jnp.float32),
        scratch_shapes=[pltpu.SemaphoreType.DMA],
        input_output_aliases={2: 0},
    )(gid, upd_rows, s_copy_table)


def kernel(x, S, W1, b1, W2, b2, W_ih, b_ih, W_hh, b_hh, Wv, bv, Wo, bo, tau):
    Wih3 = W_ih.reshape(3, D, D)
    Whh3 = W_hh.reshape(3, D, D)
    bih4 = b_ih.reshape(3, NJ, 1, BJ)
    bhh4 = b_hh.reshape(3, NJ, 1, BJ)

    s_copy, idx, w = _mean_route(x, S, W1, b1, W2, b2, tau)

    gid = (idx + NS * jnp.arange(B, dtype=jnp.int32)[:, None]).T.reshape(-1)
    hrows = _sc_gather(S.reshape(B * NS, D), gid)

    upd, mix = _gru_mix(x, hrows, Wih3, Whh3, bih4, bhh4, w.T)
    S_new = _scatter(gid, upd.reshape(K * B, D),
                     s_copy.reshape(B * NS, D)).reshape(B, NS, D)
    output = _proj(mix, Wv, bv, Wo, bo)
    return (output, S_new)


# speedup vs baseline: 2.2901x; 2.2901x over previous
import functools

import jax
import jax.numpy as jnp
from jax import lax
from jax.experimental import pallas as pl
from jax.experimental.pallas import tpu as pltpu
from jax.experimental.pallas import tpu_sc as plsc

B = 128
D = 2048
NS = 64
K = 8
H = D // 2

NW = 32
RPW = (B * K) // NW
BJ = 256
NJ = D // BJ
BB = 8
NB = B // BB

_NT = (((1,), (1,)), ((), ()))


def _dot_nt(a, b):
    return lax.dot_general(a, b, dimension_numbers=_NT,
                           preferred_element_type=jnp.float32)


def _dot_nt_bf16(a, b):
    return lax.dot_general(a.astype(jnp.bfloat16), b.astype(jnp.bfloat16),
                           dimension_numbers=_NT,
                           preferred_element_type=jnp.float32)


def _mr_body(x_ref, s_ref, w1_ref, b1_ref, w2_ref, b2_ref, tau_ref,
             idx_ref, w_ref, acc_ref):
    i = pl.program_id(0)
    acc_ref[pl.ds(i * BB, BB), :] = jnp.mean(s_ref[...], axis=1)

    @pl.when(i == NB - 1)
    def _():
        x = x_ref[...]
        m = acc_ref[...]
        w1 = w1_ref[...]
        h1 = _dot_nt(x, w1[:, :D]) + _dot_nt(m, w1[:, D:])
        h1 = h1 + b1_ref[...][None, :]
        h1 = 0.5 * h1 * (1.0 + lax.erf(h1 * (2.0 ** -0.5)))
        logits = _dot_nt(h1, w2_ref[...]) + b2_ref[...][None, :]
        logits = logits / (jnp.abs(tau_ref[0, 0]) + 0.1)

        iota = lax.broadcasted_iota(jnp.int32, (B, NS), 1)
        vals = logits
        top_v, top_i = [], []
        for _ in range(K):
            mx = jnp.max(vals, axis=1, keepdims=True)
            cand = jnp.where(vals == mx, iota, NS)
            sel = jnp.min(cand, axis=1, keepdims=True)
            top_v.append(mx)
            top_i.append(sel)
            vals = jnp.where(iota == sel, -jnp.inf, vals)
        vtop = jnp.concatenate(top_v, axis=1)
        e = jnp.exp(vtop - vtop[:, 0:1])
        w_ref[...] = e / jnp.sum(e, axis=1, keepdims=True)
        idx_ref[...] = jnp.concatenate(top_i, axis=1)


def _mean_route(x, S, W1, b1, W2, b2, tau):
    full = functools.partial(lambda n: (0,) * n)
    return pl.pallas_call(
        _mr_body,
        grid=(NB,),
        in_specs=[
            pl.BlockSpec((B, D), lambda i: (0, 0)),
            pl.BlockSpec((BB, NS, D), lambda i: (i, 0, 0)),
            pl.BlockSpec(W1.shape, lambda i: (0, 0)),
            pl.BlockSpec(b1.shape, lambda i: (0,)),
            pl.BlockSpec(W2.shape, lambda i: (0, 0)),
            pl.BlockSpec(b2.shape, lambda i: (0,)),
            pl.BlockSpec(memory_space=pltpu.SMEM),
        ],
        out_specs=(pl.BlockSpec((B, K), lambda i: (0, 0)),
                   pl.BlockSpec((B, K), lambda i: (0, 0))),
        out_shape=(jax.ShapeDtypeStruct((B, K), jnp.int32),
                   jax.ShapeDtypeStruct((B, K), jnp.float32)),
        scratch_shapes=[pltpu.VMEM((B, D), jnp.float32)],
    )(x, S, W1, b1, W2, b2, tau.reshape(1, 1))


def _sc_gather(table, gid):
    mesh = plsc.VectorSubcoreMesh(core_axis_name="c", subcore_axis_name="s")

    @functools.partial(
        pl.kernel,
        out_type=jax.ShapeDtypeStruct((B * K, D), jnp.float32),
        mesh=mesh,
        scratch_types=[
            pltpu.VMEM((RPW,), jnp.int32),
            pltpu.VMEM((RPW, D), jnp.float32),
            pltpu.SemaphoreType.DMA,
        ],
    )
    def gather_kernel(table_hbm, idx_hbm, out_hbm, idx_v, rows_v, sem):
        wid = lax.axis_index("s") * 2 + lax.axis_index("c")
        base = wid * RPW
        pltpu.sync_copy(idx_hbm.at[pl.ds(base, RPW)], idx_v)
        pltpu.async_copy(table_hbm.at[idx_v], rows_v, sem).wait()
        pltpu.sync_copy(rows_v, out_hbm.at[pl.ds(base, RPW)])

    return gather_kernel(table, gid)


def _gru_body(x_ref, h_ref, hb_ref, wih_ref, whh_ref, bih_ref, bhh_ref,
              wt_ref, upd_ref, mix_ref):
    x = x_ref[...]
    h = h_ref[...]
    wih = wih_ref[...]
    whh = whh_ref[...]

    def gi(g):
        r = _dot_nt_bf16(x, wih[g]) + bih_ref[...][g, 0, 0][None, :]
        return jnp.broadcast_to(r[None], (K, B, BJ)).reshape(K * B, BJ)

    def gh(g):
        return _dot_nt_bf16(h, whh[g]) + bhh_ref[...][g, 0, 0][None, :]

    r = jax.nn.sigmoid(gi(0) + gh(0))
    z = jax.nn.sigmoid(gi(1) + gh(1))
    n = jnp.tanh(gi(2) + r * gh(2))
    upd = (1.0 - z) * n + z * hb_ref[...]
    upd_ref[...] = upd.reshape(K, B, BJ)
    wt = wt_ref[...]
    mix_ref[...] = jnp.sum(upd.reshape(K, B, BJ) * wt[:, :, None], axis=0)


def _gru_mix(x, hrows, Wih3, Whh3, bih4, bhh4, wT):
    return pl.pallas_call(
        _gru_body,
        grid=(NJ,),
        in_specs=[
            pl.BlockSpec((B, D), lambda j: (0, 0)),
            pl.BlockSpec((K * B, D), lambda j: (0, 0)),
            pl.BlockSpec((K * B, BJ), lambda j: (0, j)),
            pl.BlockSpec((3, BJ, D), lambda j: (0, j, 0)),
            pl.BlockSpec((3, BJ, D), lambda j: (0, j, 0)),
            pl.BlockSpec((3, 1, 1, BJ), lambda j: (0, j, 0, 0)),
            pl.BlockSpec((3, 1, 1, BJ), lambda j: (0, j, 0, 0)),
            pl.BlockSpec((K, B), lambda j: (0, 0)),
        ],
        out_specs=(pl.BlockSpec((K, B, BJ), lambda j: (0, 0, j)),
                   pl.BlockSpec((B, BJ), lambda j: (0, j))),
        out_shape=(jax.ShapeDtypeStruct((K, B, D), jnp.float32),
                   jax.ShapeDtypeStruct((B, D), jnp.float32)),
    )(x, hrows, hrows, Wih3, Whh3, bih4, bhh4, wT)


def _proj_body(mix_ref, wv_ref, bv_ref, wo_ref, bo_ref, out_ref):
    v = _dot_nt_bf16(mix_ref[...], wv_ref[...]) + bv_ref[...][None, :]
    out_ref[...] = _dot_nt_bf16(v, wo_ref[...]) + bo_ref[...][None, :]


def _proj(mix, Wv, bv, Wo, bo):
    return pl.pallas_call(
        _proj_body,
        in_specs=[pl.BlockSpec(a.shape, functools.partial(lambda n: (0,) * n, a.ndim))
                  for a in (mix, Wv, bv, Wo, bo)],
        out_specs=pl.BlockSpec((B, D), lambda: (0, 0)),
        out_shape=jax.ShapeDtypeStruct((B, D), jnp.float32),
    )(mix, Wv, bv, Wo, bo)


def _merge_body(idx_sref, s_ref, upd_ref, out_ref):
    i = pl.program_id(0)
    out_ref[...] = s_ref[...]
    for bi in range(BB):
        for k in range(K):
            sl = idx_sref[i * BB + bi, k]
            out_ref[bi, pl.ds(sl, 1), :] = upd_ref[k, bi, :][None, :]


def _merge(S, upd, idx):
    return pl.pallas_call(
        _merge_body,
        grid_spec=pltpu.PrefetchScalarGridSpec(
            num_scalar_prefetch=1,
            grid=(B // BB,),
            in_specs=[
                pl.BlockSpec((BB, NS, D), lambda i, idx_s: (i, 0, 0)),
                pl.BlockSpec((K, BB, D), lambda i, idx_s: (0, i, 0)),
            ],
            out_specs=pl.BlockSpec((BB, NS, D), lambda i, idx_s: (i, 0, 0)),
        ),
        out_shape=jax.ShapeDtypeStruct((B, NS, D), jnp.float32),
    )(idx, S, upd)


def kernel(x, S, W1, b1, W2, b2, W_ih, b_ih, W_hh, b_hh, Wv, bv, Wo, bo, tau):
    Wih3 = W_ih.reshape(3, D, D)
    Whh3 = W_hh.reshape(3, D, D)
    bih4 = b_ih.reshape(3, NJ, 1, BJ)
    bhh4 = b_hh.reshape(3, NJ, 1, BJ)

    idx, w = _mean_route(x, S, W1, b1, W2, b2, tau)

    gid = (idx + NS * jnp.arange(B, dtype=jnp.int32)[:, None]).T.reshape(-1)
    hrows = _sc_gather(S.reshape(B * NS, D), gid)

    upd, mix = _gru_mix(x, hrows, Wih3, Whh3, bih4, bhh4, w.T)
    S_new = _merge(S, upd, idx)
    output = _proj(mix, Wv, bv, Wo, bo)
    return (output, S_new)
